# baseline (speedup 1.0000x reference)
import functools

import jax
import jax.numpy as jnp
from jax import lax
from jax.experimental import pallas as pl
from jax.experimental.pallas import tpu as pltpu
from jax.experimental.pallas import tpu_sc as plsc

_BATCH = 4096
_HIST = 200
_HIDDEN = 64
_ITEMS = 1000000
_CHUNK = 128
_NFULL = _ITEMS // _CHUNK
_TAIL = _ITEMS - _NFULL * _CHUNK
_SPLITS = ((0, 96), (96, 104))


def _make_transpose():
    info = plsc.get_sparse_core_info()
    nw = info.num_cores * info.num_subcores
    base_k = _NFULL // nw
    extra = _NFULL - base_k * nw
    mesh = plsc.VectorSubcoreMesh(core_axis_name="c", subcore_axis_name="s")

    @functools.partial(
        pl.kernel,
        mesh=mesh,
        out_type=jax.ShapeDtypeStruct((_ITEMS, 2 * _HIDDEN), jnp.float32),
        scratch_types=[
            pltpu.VMEM((_HIDDEN, _CHUNK), jnp.float32),
            pltpu.VMEM((_HIDDEN, _CHUNK), jnp.float32),
            pltpu.VMEM((_CHUNK, 2 * _HIDDEN), jnp.float32),
            pltpu.VMEM((_CHUNK, 2 * _HIDDEN), jnp.float32),
            pltpu.SemaphoreType.DMA,
            pltpu.SemaphoreType.DMA,
        ],
        compiler_params=pltpu.CompilerParams(needs_layout_passes=False),
    )
    def transpose_kernel(tt_hbm, tail_hbm, out_hbm, src0, src1, dst0, dst1,
                         so0, so1):
        wid = lax.axis_index("s") * info.num_cores + lax.axis_index("c")
        n_k = jnp.where(wid < extra, base_k + 1, base_k)
        rows16 = lax.iota(jnp.int32, 16)

        def transpose_block(src, dst, n_items):
            def body(i, carry):
                cols = jnp.full((16,), i, dtype=jnp.int32)
                for g in range(4):
                    v = plsc.load_gather(src, [rows16 + 16 * g, cols])
                    dst[i, pl.ds(16 * g, 16)] = v
                return carry
            lax.fori_loop(0, n_items, body, 0)

        def write(c0, dst, sem):
            return pltpu.async_copy(dst, out_hbm.at[pl.ds(c0, _CHUNK)], sem)

        def wait_write(c0, dst, sem):
            pltpu.make_async_copy(
                dst, out_hbm.at[pl.ds(c0, _CHUNK)], sem).wait()

        bufs = ((src0, dst0, so0), (src1, dst1, so1))

        def body(k, carry):
            for b in range(2):
                src, dst, so = bufs[b]
                kk = 2 * k + b

                @pl.when(kk < n_k)
                def _():
                    c0 = (wid + nw * kk) * _CHUNK
                    pltpu.sync_copy(tt_hbm.at[:, pl.ds(c0, _CHUNK)], src)

                    @pl.when(kk >= 2)
                    def _():
                        wait_write((wid + nw * (kk - 2)) * _CHUNK, dst, so)

                    transpose_block(src, dst, _CHUNK)
                    write(c0, dst, so)

            return carry

        n_outer = (base_k + 2) // 2
        lax.fori_loop(0, n_outer, body, 0)

        for b in range(2):
            for back in (2, 1):
                @pl.when((n_k >= back) & ((n_k - back) % 2 == b))
                def _(b=b, back=back):
                    wait_write((wid + nw * (n_k - back)) * _CHUNK,
                               bufs[b][1], bufs[b][2])

        @pl.when(wid == extra)
        def _():
            pltpu.sync_copy(tail_hbm, out_hbm.at[pl.ds(_NFULL * _CHUNK, _TAIL)])

    return transpose_kernel


def _make_gather():
    info = plsc.get_sparse_core_info()
    nw = info.num_cores * info.num_subcores
    rows_per_w = _BATCH // nw
    mesh = plsc.VectorSubcoreMesh(core_axis_name="c", subcore_axis_name="s")

    @functools.partial(
        pl.kernel,
        mesh=mesh,
        out_type=jax.ShapeDtypeStruct((_BATCH, _HIST, 2 * _HIDDEN), jnp.float32),
        scratch_types=[
            pltpu.VMEM((rows_per_w, _HIST), jnp.int32),
            pltpu.VMEM((_HIST, _HIDDEN), jnp.float32),
            pltpu.VMEM((_HIST, _HIDDEN), jnp.float32),
            pltpu.SemaphoreType.DMA,
            pltpu.SemaphoreType.DMA,
            pltpu.SemaphoreType.DMA,
            pltpu.SemaphoreType.DMA,
        ],
        compiler_params=pltpu.CompilerParams(use_tc_tiling_on_sc=False),
    )
    def gather_kernel(idx_hbm, table_hbm, out_hbm, idx_v, rows0, rows1,
                      si0, si1, so0, so1):
        wid = lax.axis_index("s") * info.num_cores + lax.axis_index("c")
        base = wid * rows_per_w
        pltpu.sync_copy(idx_hbm.at[pl.ds(base, rows_per_w)], idx_v)

        def fire(r, rows, sem):
            for off, width in _SPLITS:
                pltpu.async_copy(
                    table_hbm.at[idx_v.at[r, pl.ds(off, width)]],
                    rows.at[pl.ds(off, width)],
                    sem,
                )

        def drain(r, rows, sem):
            for off, width in _SPLITS:
                pltpu.make_async_copy(
                    table_hbm.at[idx_v.at[r, pl.ds(off, width)]],
                    rows.at[pl.ds(off, width)],
                    sem,
                ).wait()

        def write(r, rows, sem):
            return pltpu.async_copy(
                rows, out_hbm.at[base + r, :, pl.ds(0, _HIDDEN)], sem)

        def wait_write(r, rows, sem):
            pltpu.make_async_copy(
                rows, out_hbm.at[base + r, :, pl.ds(0, _HIDDEN)], sem).wait()

        bufs = ((rows0, si0, so0), (rows1, si1, so1))

        def body(rr, carry):
            for b in range(2):
                rows, si, so = bufs[b]
                o_rows, o_si, o_so = bufs[1 - b]
                r = 2 * rr + b

                @pl.when(r >= 2)
                def _():
                    wait_write(r - 2, rows, so)

                fire(r, rows, si)

                @pl.when(r >= 1)
                def _():
                    drain(r - 1, o_rows, o_si)
                    write(r - 1, o_rows, o_so)

            return carry

        lax.fori_loop(0, rows_per_w // 2, body, 0)

        last = rows_per_w - 1
        rows, si, so = bufs[last % 2]
        o_rows, o_si, o_so = bufs[1 - last % 2]
        drain(last, rows, si)
        write(last, rows, so)
        wait_write(last - 1, o_rows, o_so)
        wait_write(last, rows, so)

    return gather_kernel


_transpose = _make_transpose()
_gather = _make_gather()


def kernel(item_ids, table):
    idx2 = item_ids.astype(jnp.int32) * 2
    tail = jnp.pad(table[_NFULL * _CHUNK:], ((0, 0), (0, _HIDDEN)))
    tpad = _transpose(table.T, tail)
    t2 = tpad.reshape(2 * _ITEMS, _HIDDEN)
    outp = _gather(idx2, t2)
    return outp[..., :_HIDDEN]


# speedup vs baseline: 2.3943x; 2.3943x over previous
import functools

import jax
import jax.numpy as jnp
from jax import lax
from jax.experimental import pallas as pl
from jax.experimental.pallas import tpu as pltpu
from jax.experimental.pallas import tpu_sc as plsc

_BATCH = 4096
_HIST = 200
_HIDDEN = 64
_ITEMS = 1000000
_G = 2
_SPLITS = ((0, 96), (96, 104))


def _make_gather():
    info = plsc.get_sparse_core_info()
    nw = info.num_cores * info.num_subcores
    rows_per_w = _BATCH // nw
    n_steps = rows_per_w // _G
    mesh = plsc.VectorSubcoreMesh(core_axis_name="c", subcore_axis_name="s")

    @functools.partial(
        pl.kernel,
        mesh=mesh,
        out_type=jax.ShapeDtypeStruct((_BATCH, _HIST, 2 * _HIDDEN), jnp.float32),
        scratch_types=[
            pltpu.VMEM((rows_per_w, _HIST), jnp.int32),
            pltpu.VMEM((_G, _HIST, _HIDDEN), jnp.float32),
            pltpu.VMEM((_G, _HIST, _HIDDEN), jnp.float32),
            pltpu.SemaphoreType.DMA,
            pltpu.SemaphoreType.DMA,
            pltpu.SemaphoreType.DMA,
            pltpu.SemaphoreType.DMA,
        ],
        compiler_params=pltpu.CompilerParams(use_tc_tiling_on_sc=False),
    )
    def gather_kernel(idx_hbm, table_hbm, out_hbm, idx_v, rows0, rows1,
                      si0, si1, so0, so1):
        wid = lax.axis_index("s") * info.num_cores + lax.axis_index("c")
        base = wid * rows_per_w
        pltpu.sync_copy(idx_hbm.at[pl.ds(base, rows_per_w)], idx_v)

        def fire(c, rows, sem):
            for j in range(_G):
                for off, width in _SPLITS:
                    pltpu.async_copy(
                        table_hbm.at[idx_v.at[_G * c + j, pl.ds(off, width)]],
                        rows.at[j, pl.ds(off, width)],
                        sem,
                    )

        def drain(c, rows, sem):
            for j in range(_G):
                for off, width in _SPLITS:
                    pltpu.make_async_copy(
                        table_hbm.at[idx_v.at[_G * c + j, pl.ds(off, width)]],
                        rows.at[j, pl.ds(off, width)],
                        sem,
                    ).wait()

        def write(c, rows, sem):
            return pltpu.async_copy(
                rows,
                out_hbm.at[pl.ds(base + _G * c, _G), :, pl.ds(0, _HIDDEN)],
                sem)

        def wait_write(c, rows, sem):
            pltpu.make_async_copy(
                rows,
                out_hbm.at[pl.ds(base + _G * c, _G), :, pl.ds(0, _HIDDEN)],
                sem).wait()

        bufs = ((rows0, si0, so0), (rows1, si1, so1))

        def body(cc, carry):
            for b in range(2):
                rows, si, so = bufs[b]
                o_rows, o_si, o_so = bufs[1 - b]
                c = 2 * cc + b

                @pl.when(c >= 2)
                def _():
                    wait_write(c - 2, rows, so)

                fire(c, rows, si)

                @pl.when(c >= 1)
                def _():
                    drain(c - 1, o_rows, o_si)
                    write(c - 1, o_rows, o_so)

            return carry

        lax.fori_loop(0, n_steps // 2, body, 0)

        last = n_steps - 1
        rows, si, so = bufs[last % 2]
        o_rows, o_si, o_so = bufs[1 - last % 2]
        drain(last, rows, si)
        write(last, rows, so)
        wait_write(last - 1, o_rows, o_so)
        wait_write(last, rows, so)

    return gather_kernel


_gather = _make_gather()


def kernel(item_ids, table):
    idx2 = item_ids.astype(jnp.int32) * 2
    tpad = jnp.pad(table, ((0, 0), (0, _HIDDEN)))
    t2 = tpad.reshape(2 * _ITEMS, _HIDDEN)
    outp = _gather(idx2, t2)
    return outp[..., :_HIDDEN]


# speedup vs baseline: 2.5695x; 1.0732x over previous
import functools

import jax
import jax.numpy as jnp
from jax import lax
from jax.experimental import pallas as pl
from jax.experimental.pallas import tpu as pltpu
from jax.experimental.pallas import tpu_sc as plsc

_BATCH = 4096
_HIST = 200
_HIDDEN = 64
_ITEMS = 1000000
_G = 2
_SPLITS = ((0, 96), (96, 104))
_TPR = 2048
_NMAIN = (_ITEMS // _TPR) * _TPR
_NTAIL = _ITEMS - _NMAIN


def _tp_body(tt_ref, tail_ref, out_ref):
    x = tt_ref[...]
    out_ref[:, :_HIDDEN] = x.T
    out_ref[:, _HIDDEN:] = jnp.zeros((_TPR, _HIDDEN), jnp.float32)

    @pl.when(pl.program_id(0) == 0)
    def _():
        out_ref[:_NTAIL, _HIDDEN:] = tail_ref[...].T


_tp = pl.pallas_call(
    _tp_body,
    grid=(_ITEMS // _TPR,),
    in_specs=[
        pl.BlockSpec((_HIDDEN, _TPR), lambda i: (0, i)),
        pl.BlockSpec((_HIDDEN, _NTAIL), lambda i: (0, 0)),
    ],
    out_specs=pl.BlockSpec((_TPR, 2 * _HIDDEN), lambda i: (i, 0)),
    out_shape=jax.ShapeDtypeStruct((_NMAIN, 2 * _HIDDEN), jnp.float32),
)


def _make_gather():
    info = plsc.get_sparse_core_info()
    nw = info.num_cores * info.num_subcores
    rows_per_w = _BATCH // nw
    n_steps = rows_per_w // _G
    mesh = plsc.VectorSubcoreMesh(core_axis_name="c", subcore_axis_name="s")

    @functools.partial(
        pl.kernel,
        mesh=mesh,
        out_type=jax.ShapeDtypeStruct((_BATCH, _HIST, 2 * _HIDDEN), jnp.float32),
        scratch_types=[
            pltpu.VMEM((rows_per_w, _HIST), jnp.int32),
            pltpu.VMEM((_G, _HIST, _HIDDEN), jnp.float32),
            pltpu.VMEM((_G, _HIST, _HIDDEN), jnp.float32),
            pltpu.SemaphoreType.DMA,
            pltpu.SemaphoreType.DMA,
            pltpu.SemaphoreType.DMA,
            pltpu.SemaphoreType.DMA,
        ],
        compiler_params=pltpu.CompilerParams(use_tc_tiling_on_sc=False),
    )
    def gather_kernel(idx_hbm, table_hbm, out_hbm, idx_v, rows0, rows1,
                      si0, si1, so0, so1):
        wid = lax.axis_index("s") * info.num_cores + lax.axis_index("c")
        base = wid * rows_per_w
        pltpu.sync_copy(idx_hbm.at[pl.ds(base, rows_per_w)], idx_v)

        def fire(c, rows, sem):
            for j in range(_G):
                for off, width in _SPLITS:
                    pltpu.async_copy(
                        table_hbm.at[idx_v.at[_G * c + j, pl.ds(off, width)]],
                        rows.at[j, pl.ds(off, width)],
                        sem,
                    )

        def drain(c, rows, sem):
            for j in range(_G):
                for off, width in _SPLITS:
                    pltpu.make_async_copy(
                        table_hbm.at[idx_v.at[_G * c + j, pl.ds(off, width)]],
                        rows.at[j, pl.ds(off, width)],
                        sem,
                    ).wait()

        def write(c, rows, sem):
            return pltpu.async_copy(
                rows,
                out_hbm.at[pl.ds(base + _G * c, _G), :, pl.ds(0, _HIDDEN)],
                sem)

        def wait_write(c, rows, sem):
            pltpu.make_async_copy(
                rows,
                out_hbm.at[pl.ds(base + _G * c, _G), :, pl.ds(0, _HIDDEN)],
                sem).wait()

        bufs = ((rows0, si0, so0), (rows1, si1, so1))

        def body(cc, carry):
            for b in range(2):
                rows, si, so = bufs[b]
                o_rows, o_si, o_so = bufs[1 - b]
                c = 2 * cc + b

                @pl.when(c >= 2)
                def _():
                    wait_write(c - 2, rows, so)

                fire(c, rows, si)

                @pl.when(c >= 1)
                def _():
                    drain(c - 1, o_rows, o_si)
                    write(c - 1, o_rows, o_so)

            return carry

        lax.fori_loop(0, n_steps // 2, body, 0)

        last = n_steps - 1
        rows, si, so = bufs[last % 2]
        o_rows, o_si, o_so = bufs[1 - last % 2]
        drain(last, rows, si)
        write(last, rows, so)
        wait_write(last - 1, o_rows, o_so)
        wait_write(last, rows, so)

    return gather_kernel


_gather = _make_gather()


def kernel(item_ids, table):
    ids = item_ids.astype(jnp.int32)
    idx2 = jnp.where(ids < _NMAIN, 2 * ids, 2 * (ids - _NMAIN) + 1)
    tail_t = table.T[:, _NMAIN:]
    tpad = _tp(table.T, tail_t)
    t2 = tpad.reshape(2 * _NMAIN, _HIDDEN)
    outp = _gather(idx2, t2)
    return outp[..., :_HIDDEN]
